# Initial kernel scaffold; baseline (speedup 1.0000x reference)
import jax
import jax.numpy as jnp
from jax.experimental import pallas as pl


def kernel(metadata_entry, brand_entry, category_entry, price_entry, user_product_match_entry, program_types_input, W_meta, W_brand, W_cat, W_upm, W_dense, b_dense):
    raise NotImplementedError("write your pallas kernel here")


# speedup vs baseline: 7.1847x; 7.1847x over previous
import functools

import jax
import jax.numpy as jnp
from jax import lax
from jax.experimental import pallas as pl
from jax.experimental.pallas import tpu as pltpu
from jax.experimental.pallas import tpu_sc as plsc

B, L, T, D = 1024, 50, 20, 64
N = B * L
NC, NS = 2, 16
NW = NC * NS
POS_PER_W = N // NW
CHUNK = 32
N_CHUNKS = POS_PER_W // CHUNK
IDX_ROWS = (CHUNK * T) // 128
MIDX_ROWS_PER_W = POS_PER_W * T // 128


def _sc_body(wmeta, wbrand, wcat, midx, bidx, cidx, s_out,
             midx_v, bidx_v, cidx_v, gbuf, bbuf, cbuf, sbuf, sem):
    wid = lax.axis_index("s") * NC + lax.axis_index("c")
    base = wid * POS_PER_W
    chunk_base = wid * N_CHUNKS

    def chunk_body(g, carry):
        p0 = base + g * CHUNK
        pltpu.sync_copy(midx.at[chunk_base + g], midx_v)
        pltpu.sync_copy(bidx.at[pl.ds(p0, CHUNK)], bidx_v)
        pltpu.sync_copy(cidx.at[pl.ds(p0, CHUNK)], cidx_v)
        cps = []
        for j in range(IDX_ROWS):
            cps.append(pltpu.async_copy(
                wmeta.at[midx_v.at[j]], gbuf.at[pl.ds(j * 128, 128)], sem))
        cps.append(pltpu.async_copy(wbrand.at[bidx_v], bbuf, sem))
        cps.append(pltpu.async_copy(wcat.at[cidx_v], cbuf, sem))
        for cp in cps:
            cp.wait()

        def pos_body(i, carry2):
            base_row = i * T
            for v in range(D // 16):
                col = pl.ds(v * 16, 16)
                acc = gbuf[base_row, col]
                for t in range(1, T):
                    acc = acc + gbuf[base_row + t, col]
                sbuf[i, col] = acc * (1.0 / T) + bbuf[i, col] + cbuf[i, col]
            return carry2

        lax.fori_loop(0, CHUNK, pos_body, 0)
        pltpu.sync_copy(sbuf, s_out.at[pl.ds(p0, CHUNK)])
        return carry

    lax.fori_loop(0, N_CHUNKS, chunk_body, 0)


@jax.jit
def _sc_gather_sum(wmeta, wbrand, wcat, midx, bidx, cidx):
    mesh = plsc.VectorSubcoreMesh(core_axis_name="c", subcore_axis_name="s")
    return pl.kernel(
        _sc_body,
        out_type=jax.ShapeDtypeStruct((N, D), jnp.float32),
        mesh=mesh,
        compiler_params=pltpu.CompilerParams(use_tc_tiling_on_sc=False),
        scratch_types=[
            pltpu.VMEM((IDX_ROWS, 128), jnp.int32),
            pltpu.VMEM((CHUNK,), jnp.int32),
            pltpu.VMEM((CHUNK,), jnp.int32),
            pltpu.VMEM((CHUNK * T, D), jnp.float32),
            pltpu.VMEM((CHUNK, D), jnp.float32),
            pltpu.VMEM((CHUNK, D), jnp.float32),
            pltpu.VMEM((CHUNK, D), jnp.float32),
            pltpu.SemaphoreType.DMA,
        ],
    )(wmeta, wbrand, wcat, midx, bidx, cidx)


TC_R = 1024


def _tc_body(s_ref, price_ref, prog_ref, upm_ref, w2_ref, woh_ref,
             wupm_ref, r_ref, b_ref, o_ref):
    sv = s_ref[:]
    upm = upm_ref[:]
    prog = prog_ref[:]
    iota128 = lax.broadcasted_iota(jnp.int32, (TC_R, 128), 1)
    ohu = (upm == iota128).astype(jnp.float32)
    iota32 = lax.broadcasted_iota(jnp.int32, (TC_R, 32), 1)
    ohp = (prog == iota32).astype(jnp.float32)
    upm_emb = jnp.dot(ohu, wupm_ref[:], preferred_element_type=jnp.float32)
    x = jnp.concatenate([sv, upm_emb], axis=1)
    acc = jnp.dot(x, w2_ref[:], preferred_element_type=jnp.float32)
    acc += jnp.dot(ohp, woh_ref[:], preferred_element_type=jnp.float32)
    acc += price_ref[:] * r_ref[:]
    acc += b_ref[:]
    o_ref[:] = acc


@jax.jit
def _tc_dense(s, price, prog, upm, w2, woh, wupm_pad, r, b):
    grid = (N // TC_R,)
    return pl.pallas_call(
        _tc_body,
        grid=grid,
        in_specs=[
            pl.BlockSpec((TC_R, D), lambda i: (i, 0)),
            pl.BlockSpec((TC_R, 1), lambda i: (i, 0)),
            pl.BlockSpec((TC_R, 1), lambda i: (i, 0)),
            pl.BlockSpec((TC_R, 1), lambda i: (i, 0)),
            pl.BlockSpec((128, D), lambda i: (0, 0)),
            pl.BlockSpec((32, D), lambda i: (0, 0)),
            pl.BlockSpec((128, D), lambda i: (0, 0)),
            pl.BlockSpec((1, D), lambda i: (0, 0)),
            pl.BlockSpec((1, D), lambda i: (0, 0)),
        ],
        out_specs=pl.BlockSpec((TC_R, D), lambda i: (i, 0)),
        out_shape=jax.ShapeDtypeStruct((N, D), jnp.float32),
    )(s, price, prog, upm, w2, woh, wupm_pad, r, b)


def kernel(metadata_entry, brand_entry, category_entry, price_entry,
           user_product_match_entry, program_types_input,
           W_meta, W_brand, W_cat, W_upm, W_dense, b_dense):
    midx = metadata_entry.astype(jnp.int32).reshape(
        NW * N_CHUNKS, IDX_ROWS, 128)
    bidx = brand_entry.astype(jnp.int32).reshape(N)
    cidx = category_entry.astype(jnp.int32).reshape(N)
    s = _sc_gather_sum(W_meta, W_brand, W_cat, midx, bidx, cidx)

    price = price_entry.astype(jnp.float32).reshape(N, 1)
    prog = program_types_input.astype(jnp.int32).reshape(N, 1)
    upm = user_product_match_entry.astype(jnp.int32).reshape(N, 1)
    w2 = jnp.concatenate([W_dense[:D], W_dense[D + 1 + 32:]], axis=0)
    woh = W_dense[D + 1:D + 1 + 32]
    r = W_dense[D:D + 1]
    wupm_pad = jnp.zeros((128, D), jnp.float32).at[:W_upm.shape[0]].set(W_upm)
    out = _tc_dense(s, price, prog, upm, w2, woh, wupm_pad, r,
                    b_dense.reshape(1, D))
    return out.reshape(B, L, D)


# speedup vs baseline: 9.9359x; 1.3829x over previous
import functools

import jax
import jax.numpy as jnp
from jax import lax
from jax.experimental import pallas as pl
from jax.experimental.pallas import tpu as pltpu
from jax.experimental.pallas import tpu_sc as plsc

B, L, T, D = 1024, 50, 20, 64
N = B * L
NC, NS = 2, 16
NW = NC * NS
POS_PER_W = N // NW
CHUNK = 32
N_CHUNKS = POS_PER_W // CHUNK
IDX_ROWS = (CHUNK * T) // 128
MIDX_ROWS_PER_W = POS_PER_W * T // 128


def _sc_body(wmeta, wbrand, wcat, midx, bidx, cidx, s_out,
             midx_v, bidx_all, cidx_all, gbuf, bbuf, cbuf, sbuf,
             sem_g, sem_i, sem_o):
    wid = lax.axis_index("s") * NC + lax.axis_index("c")
    base = wid * POS_PER_W
    chunk_base = wid * N_CHUNKS

    def fire_chunk(g, p):
        for j in range(IDX_ROWS):
            pltpu.async_copy(
                wmeta.at[midx_v.at[p, j]],
                gbuf.at[p].at[pl.ds(j * 128, 128)], sem_g)
        loc = pl.ds((g - chunk_base) * CHUNK, CHUNK)
        pltpu.async_copy(wbrand.at[bidx_all.at[loc]], bbuf.at[p], sem_g)
        pltpu.async_copy(wcat.at[cidx_all.at[loc]], cbuf.at[p], sem_g)

    def wait_chunk(p):
        for j in range(IDX_ROWS):
            pltpu.make_async_copy(
                wmeta.at[midx_v.at[p, j]],
                gbuf.at[p].at[pl.ds(j * 128, 128)], sem_g).wait()
        pltpu.make_async_copy(wbrand.at[bidx_all.at[pl.ds(0, CHUNK)]],
                              bbuf.at[p], sem_g).wait()
        pltpu.make_async_copy(wcat.at[cidx_all.at[pl.ds(0, CHUNK)]],
                              cbuf.at[p], sem_g).wait()

    pltpu.sync_copy(bidx.at[pl.ds(base, POS_PER_W)], bidx_all)
    pltpu.sync_copy(cidx.at[pl.ds(base, POS_PER_W)], cidx_all)
    pltpu.sync_copy(midx.at[chunk_base], midx_v.at[0])
    fire_chunk(chunk_base, 0)
    pltpu.async_copy(midx.at[chunk_base + 1], midx_v.at[1], sem_i)

    def outer_body(gg, carry):
        for b in range(2):
            g = gg * 2 + b
            p, np_ = b, 1 - b
            wait_chunk(p)

            @pl.when(g < N_CHUNKS - 1)
            def _():
                pltpu.make_async_copy(
                    midx.at[chunk_base + 1], midx_v.at[np_], sem_i).wait()
                fire_chunk(chunk_base + g + 1, np_)

            @pl.when(g < N_CHUNKS - 2)
            def _():
                pltpu.async_copy(
                    midx.at[chunk_base + g + 2], midx_v.at[p], sem_i)

            @pl.when(g >= 2)
            def _():
                pltpu.make_async_copy(
                    sbuf.at[p], s_out.at[pl.ds(base, CHUNK)], sem_o).wait()

            def pos_body(i, carry2):
                base_row = i * T
                for v in range(D // 16):
                    col = pl.ds(v * 16, 16)
                    acc = gbuf[p, base_row, col]
                    for t in range(1, T):
                        acc = acc + gbuf[p, base_row + t, col]
                    sbuf[p, i, col] = (acc * (1.0 / T)
                                       + bbuf[p, i, col] + cbuf[p, i, col])
                return carry2

            lax.fori_loop(0, CHUNK, pos_body, 0)
            pltpu.async_copy(
                sbuf.at[p], s_out.at[pl.ds(base + g * CHUNK, CHUNK)], sem_o)
        return carry

    lax.fori_loop(0, N_CHUNKS // 2, outer_body, 0)
    for _ in range(2):
        pltpu.make_async_copy(
            sbuf.at[0], s_out.at[pl.ds(base, CHUNK)], sem_o).wait()


@jax.jit
def _sc_gather_sum(wmeta, wbrand, wcat, midx, bidx, cidx):
    mesh = plsc.VectorSubcoreMesh(core_axis_name="c", subcore_axis_name="s")
    return pl.kernel(
        _sc_body,
        out_type=jax.ShapeDtypeStruct((N, D), jnp.float32),
        mesh=mesh,
        compiler_params=pltpu.CompilerParams(use_tc_tiling_on_sc=False),
        scratch_types=[
            pltpu.VMEM((2, IDX_ROWS, 128), jnp.int32),
            pltpu.VMEM((POS_PER_W,), jnp.int32),
            pltpu.VMEM((POS_PER_W,), jnp.int32),
            pltpu.VMEM((2, CHUNK * T, D), jnp.float32),
            pltpu.VMEM((2, CHUNK, D), jnp.float32),
            pltpu.VMEM((2, CHUNK, D), jnp.float32),
            pltpu.VMEM((2, CHUNK, D), jnp.float32),
            pltpu.SemaphoreType.DMA,
            pltpu.SemaphoreType.DMA,
            pltpu.SemaphoreType.DMA,
        ],
    )(wmeta, wbrand, wcat, midx, bidx, cidx)


TC_R = 1024


def _tc_body(s_ref, price_ref, prog_ref, upm_ref, w2_ref, woh_ref,
             wupm_ref, r_ref, b_ref, o_ref):
    sv = s_ref[:]
    upm = upm_ref[:]
    prog = prog_ref[:]
    iota128 = lax.broadcasted_iota(jnp.int32, (TC_R, 128), 1)
    ohu = (upm == iota128).astype(jnp.float32)
    iota32 = lax.broadcasted_iota(jnp.int32, (TC_R, 32), 1)
    ohp = (prog == iota32).astype(jnp.float32)
    upm_emb = jnp.dot(ohu, wupm_ref[:], preferred_element_type=jnp.float32)
    x = jnp.concatenate([sv, upm_emb], axis=1)
    acc = jnp.dot(x, w2_ref[:], preferred_element_type=jnp.float32)
    acc += jnp.dot(ohp, woh_ref[:], preferred_element_type=jnp.float32)
    acc += price_ref[:] * r_ref[:]
    acc += b_ref[:]
    o_ref[:] = acc


@jax.jit
def _tc_dense(s, price, prog, upm, w2, woh, wupm_pad, r, b):
    grid = (N // TC_R,)
    return pl.pallas_call(
        _tc_body,
        grid=grid,
        in_specs=[
            pl.BlockSpec((TC_R, D), lambda i: (i, 0)),
            pl.BlockSpec((TC_R, 1), lambda i: (i, 0)),
            pl.BlockSpec((TC_R, 1), lambda i: (i, 0)),
            pl.BlockSpec((TC_R, 1), lambda i: (i, 0)),
            pl.BlockSpec((128, D), lambda i: (0, 0)),
            pl.BlockSpec((32, D), lambda i: (0, 0)),
            pl.BlockSpec((128, D), lambda i: (0, 0)),
            pl.BlockSpec((1, D), lambda i: (0, 0)),
            pl.BlockSpec((1, D), lambda i: (0, 0)),
        ],
        out_specs=pl.BlockSpec((TC_R, D), lambda i: (i, 0)),
        out_shape=jax.ShapeDtypeStruct((N, D), jnp.float32),
    )(s, price, prog, upm, w2, woh, wupm_pad, r, b)


def kernel(metadata_entry, brand_entry, category_entry, price_entry,
           user_product_match_entry, program_types_input,
           W_meta, W_brand, W_cat, W_upm, W_dense, b_dense):
    midx = metadata_entry.astype(jnp.int32).reshape(
        NW * N_CHUNKS, IDX_ROWS, 128)
    bidx = brand_entry.astype(jnp.int32).reshape(N)
    cidx = category_entry.astype(jnp.int32).reshape(N)
    s = _sc_gather_sum(W_meta, W_brand, W_cat, midx, bidx, cidx)

    price = price_entry.astype(jnp.float32).reshape(N, 1)
    prog = program_types_input.astype(jnp.int32).reshape(N, 1)
    upm = user_product_match_entry.astype(jnp.int32).reshape(N, 1)
    w2 = jnp.concatenate([W_dense[:D], W_dense[D + 1 + 32:]], axis=0)
    woh = W_dense[D + 1:D + 1 + 32]
    r = W_dense[D:D + 1]
    wupm_pad = jnp.zeros((128, D), jnp.float32).at[:W_upm.shape[0]].set(W_upm)
    out = _tc_dense(s, price, prog, upm, w2, woh, wupm_pad, r,
                    b_dense.reshape(1, D))
    return out.reshape(B, L, D)
